# baseline (speedup 1.0000x reference)
import functools

import jax
import jax.numpy as jnp
from jax import lax
from jax.experimental import pallas as pl
from jax.experimental.pallas import tpu as pltpu
from jax.experimental.pallas import tpu_sc as plsc

_T = 4096
_D = 768
_K = 512
_C = 20
_CD = _C + 1
_BLK = 512
_NBLK = _T // _BLK
_KS = 9
_PAD = _KS // 2
_MOM = 0.95
_CONF_TH = 0.7

_NC = 2
_NS = 16
_NW = _NC * _NS
_RW = _T // _NW


def _stage1_body(feat_ref, w_ref, avg_ref, proj_ref, preds_ref):
    x = feat_ref[...]
    f = x / jnp.maximum(jnp.sqrt(jnp.sum(x * x, axis=1, keepdims=True)), 1e-12)
    p = lax.dot_general(f, w_ref[...], (((1,), (1,)), ((), ())),
                        preferred_element_type=jnp.float32)
    pn = p / jnp.maximum(jnp.sqrt(jnp.sum(p * p, axis=1, keepdims=True)), 1e-12)
    proj_ref[...] = pn

    sims = lax.dot_general(pn, avg_ref[...], (((1,), (1,)), ((), ())),
                           preferred_element_type=jnp.float32)
    m = jnp.max(sims, axis=1, keepdims=True)
    e = jnp.exp((sims - m) / 0.1)
    s = jnp.sum(e, axis=1, keepdims=True)
    conf = e / s
    max_conf = jnp.max(conf, axis=1)
    preds = jnp.argmax(conf, axis=1).astype(jnp.int32)
    mask = max_conf > _CONF_TH
    preds_ref[...] = jnp.where(mask, preds, _C).reshape(1, 1, _BLK)


def _cov_body(feat_ref, cov_ref):
    i = pl.program_id(0)
    x = feat_ref[...]
    f = x / jnp.maximum(jnp.sqrt(jnp.sum(x * x, axis=1, keepdims=True)), 1e-12)

    @pl.when(i == 0)
    def _init():
        cov_ref[...] = jnp.zeros_like(cov_ref)

    cov_ref[...] += lax.dot_general(f[:, :_K], f, (((0,), (0,)), ((), ())),
                                    preferred_element_type=jnp.float32)


@functools.partial(
    pl.kernel,
    mesh=plsc.VectorSubcoreMesh(core_axis_name="c", subcore_axis_name="s"),
    out_type=jax.ShapeDtypeStruct((2 * _NW, _CD * _K), jnp.float32),
    scratch_types=[
        pltpu.VMEM((_RW, _K), jnp.float32),
        pltpu.VMEM((_RW,), jnp.int32),
        pltpu.VMEM((_CD * _K,), jnp.float32),
        pltpu.VMEM((_CD * _K,), jnp.float32),
    ],
    compiler_params=pltpu.CompilerParams(needs_layout_passes=False),
)
def _sc_segsum(proj_hbm, preds_hbm, zsum_hbm, sums_hbm,
               rows_v, idx_v, acc_a, acc_b):
    cid = lax.axis_index("c")
    sid = lax.axis_index("s")
    wid = cid * _NS + sid
    base = wid * _RW

    pltpu.sync_copy(preds_hbm.at[pl.ds(base, _RW)], idx_v)
    pltpu.sync_copy(proj_hbm.at[pl.ds(base, _RW)], rows_v)
    pltpu.sync_copy(zsum_hbm, acc_a)
    pltpu.sync_copy(zsum_hbm, acc_b)

    lanes = lax.iota(jnp.int32, 16)
    coffs = [lanes + (j * 16) for j in range(_K // 16)]
    _UNR = 16

    def body(step, carry):
        r0 = step * _UNR
        for u in range(_UNR):
            r = r0 + u
            pv = plsc.load_gather(idx_v, [jnp.full((16,), r, jnp.int32)])
            pbase = pv * _K
            acc = acc_a if u % 2 == 0 else acc_b
            for j in range(_K // 16):
                vals = rows_v[r, pl.ds(j * 16, 16)]
                plsc.addupdate_scatter(acc, [pbase + coffs[j]], vals)
        return carry

    lax.fori_loop(0, _RW // _UNR, body, 0)

    pltpu.sync_copy(acc_a, sums_hbm.at[2 * wid])
    pltpu.sync_copy(acc_b, sums_hbm.at[2 * wid + 1])


def _stage2_body(proj_ref, sums_ref, preds_ref, avg_ref, w_ref, cov_ref,
                 sm_ref, ids_ref, newproj_ref, pad_ref):
    sums = sums_ref[0, :_C, :]
    for w in range(1, 2 * _NW):
        sums = sums + sums_ref[w, :_C, :]
    cls = lax.broadcasted_iota(jnp.int32, (_C, 1), 0)
    counts_col = jnp.sum((preds_ref[...] == cls).astype(jnp.float32),
                         axis=1, keepdims=True)
    avg = avg_ref[...]

    means = sums / jnp.maximum(counts_col, 1.0)
    upd = _MOM * avg + (1.0 - _MOM) * means
    updn = upd / jnp.maximum(jnp.sqrt(jnp.sum(upd * upd, axis=1, keepdims=True)), 1e-12)
    new_avg = jnp.where(counts_col > 0.0, updn, avg)

    simil = lax.dot_general(proj_ref[...], new_avg, (((1,), (1,)), ((), ())),
                            preferred_element_type=jnp.float32)
    class_sims = jnp.max(simil, axis=1)
    ids_ref[...] = jnp.argmax(simil, axis=1).astype(jnp.int32).reshape(1, _T)

    pad_ref[0, pl.ds(_PAD, _T)] = class_sims
    for k in range(_PAD):
        pad_ref[0, k:k + 1] = pad_ref[0, 2 * _PAD - k:2 * _PAD - k + 1]
        pad_ref[0, _PAD + _T + k:_PAD + _T + k + 1] = \
            pad_ref[0, _PAD + _T - 2 - k:_PAD + _T - 1 - k]
    acc = pad_ref[0, pl.ds(0, _T)]
    for j in range(1, _KS):
        acc = acc + pad_ref[0, pl.ds(j, _T)]
    sm_ref[...] = (acc * (1.0 / _KS)).reshape(1, _T)

    any_mask = jnp.sum(counts_col) > 0.0
    newproj_ref[...] = jnp.where(any_mask,
                                 _MOM * w_ref[...] + (1.0 - _MOM) * cov_ref[...],
                                 w_ref[...])


@jax.jit
def kernel(features, proj_w, avg_features):
    proj, preds3d = pl.pallas_call(
        _stage1_body,
        grid=(_NBLK,),
        in_specs=[
            pl.BlockSpec((_BLK, _D), lambda i: (i, 0)),
            pl.BlockSpec((_K, _D), lambda i: (0, 0)),
            pl.BlockSpec((_C, _K), lambda i: (0, 0)),
        ],
        out_specs=[
            pl.BlockSpec((_BLK, _K), lambda i: (i, 0)),
            pl.BlockSpec((1, 1, _BLK), lambda i: (i, 0, 0)),
        ],
        out_shape=[
            jax.ShapeDtypeStruct((_T, _K), jnp.float32),
            jax.ShapeDtypeStruct((_NBLK, 1, _BLK), jnp.int32),
        ],
    )(features, proj_w, avg_features)

    preds = preds3d.reshape(_T)
    zsum = jnp.zeros((_CD * _K,), jnp.float32)
    sums = _sc_segsum(proj, preds, zsum).reshape(2 * _NW, _CD, _K)

    cov = pl.pallas_call(
        _cov_body,
        grid=(_NBLK,),
        in_specs=[pl.BlockSpec((_BLK, _D), lambda i: (i, 0))],
        out_specs=pl.BlockSpec((_K, _D), lambda i: (0, 0)),
        out_shape=jax.ShapeDtypeStruct((_K, _D), jnp.float32),
    )(features)

    smoothed, class_ids, new_proj = pl.pallas_call(
        _stage2_body,
        out_shape=[
            jax.ShapeDtypeStruct((1, _T), jnp.float32),
            jax.ShapeDtypeStruct((1, _T), jnp.int32),
            jax.ShapeDtypeStruct((_K, _D), jnp.float32),
        ],
        scratch_shapes=[pltpu.VMEM((1, _T + 2 * _PAD), jnp.float32)],
    )(proj, sums, preds.reshape(1, _T), avg_features, proj_w, cov)

    return smoothed.reshape(_T), class_ids.reshape(_T), new_proj


# speedup vs baseline: 1.0780x; 1.0780x over previous
import functools

import jax
import jax.numpy as jnp
from jax import lax
from jax.experimental import pallas as pl
from jax.experimental.pallas import tpu as pltpu
from jax.experimental.pallas import tpu_sc as plsc

_T = 4096
_D = 768
_K = 512
_C = 20
_CD = _C + 1
_BLK = 512
_NBLK = _T // _BLK
_KS = 9
_PAD = _KS // 2
_MOM = 0.95
_CONF_TH = 0.7

_NC = 2
_NS = 16
_NW = _NC * _NS
_RW = _T // _NW


def _stage1_body(feat_ref, w_ref, avg_ref, proj_ref, preds_ref):
    x = feat_ref[...]
    f = x / jnp.maximum(jnp.sqrt(jnp.sum(x * x, axis=1, keepdims=True)), 1e-12)
    p = lax.dot_general(f, w_ref[...], (((1,), (1,)), ((), ())),
                        preferred_element_type=jnp.float32)
    pn = p / jnp.maximum(jnp.sqrt(jnp.sum(p * p, axis=1, keepdims=True)), 1e-12)
    proj_ref[...] = pn

    sims = lax.dot_general(pn, avg_ref[...], (((1,), (1,)), ((), ())),
                           preferred_element_type=jnp.float32)
    m = jnp.max(sims, axis=1, keepdims=True)
    e = jnp.exp((sims - m) / 0.1)
    s = jnp.sum(e, axis=1, keepdims=True)
    conf = e / s
    max_conf = jnp.max(conf, axis=1)
    preds = jnp.argmax(conf, axis=1).astype(jnp.int32)
    mask = max_conf > _CONF_TH
    preds_ref[...] = jnp.where(mask, preds, _C).reshape(1, 1, _BLK)


def _cov_body(feat_ref, cov_ref):
    i = pl.program_id(0)
    x = feat_ref[...]
    f = x / jnp.maximum(jnp.sqrt(jnp.sum(x * x, axis=1, keepdims=True)), 1e-12)

    @pl.when(i == 0)
    def _init():
        cov_ref[...] = jnp.zeros_like(cov_ref)

    cov_ref[...] += lax.dot_general(f[:, :_K], f, (((0,), (0,)), ((), ())),
                                    preferred_element_type=jnp.float32)


@functools.partial(
    pl.kernel,
    mesh=plsc.VectorSubcoreMesh(core_axis_name="c", subcore_axis_name="s"),
    out_type=jax.ShapeDtypeStruct((_NW, _CD * _K), jnp.float32),
    scratch_types=[
        pltpu.VMEM((_RW, _K), jnp.float32),
        pltpu.VMEM((_RW,), jnp.int32),
        pltpu.VMEM((_CD * _K,), jnp.float32),
    ],
    compiler_params=pltpu.CompilerParams(needs_layout_passes=False),
)
def _sc_segsum(proj_hbm, preds_hbm, zsum_hbm, sums_hbm, rows_v, idx_v, acc_v):
    cid = lax.axis_index("c")
    sid = lax.axis_index("s")
    wid = cid * _NS + sid
    base = wid * _RW

    pltpu.sync_copy(preds_hbm.at[pl.ds(base, _RW)], idx_v)
    pltpu.sync_copy(proj_hbm.at[pl.ds(base, _RW)], rows_v)
    pltpu.sync_copy(zsum_hbm, acc_v)

    lanes = lax.iota(jnp.int32, 16)
    coffs = [lanes + (j * 16) for j in range(_K // 16)]
    _UNR = 16

    def body(step, carry):
        r0 = step * _UNR
        for u in range(_UNR):
            r = r0 + u
            pv = plsc.load_gather(idx_v, [jnp.full((16,), r, jnp.int32)])
            pbase = pv * _K
            for j in range(_K // 16):
                vals = rows_v[r, pl.ds(j * 16, 16)]
                plsc.addupdate_scatter(acc_v, [pbase + coffs[j]], vals)
        return carry

    lax.fori_loop(0, _RW // _UNR, body, 0)

    pltpu.sync_copy(acc_v, sums_hbm.at[wid])


def _stage2_body(proj_ref, sums_ref, preds_ref, avg_ref, w_ref, cov_ref,
                 sm_ref, ids_ref, newproj_ref, pad_ref):
    sums = sums_ref[0, :_C, :]
    for w in range(1, _NW):
        sums = sums + sums_ref[w, :_C, :]
    cls = lax.broadcasted_iota(jnp.int32, (_C, 1), 0)
    counts_col = jnp.sum((preds_ref[...] == cls).astype(jnp.float32),
                         axis=1, keepdims=True)
    avg = avg_ref[...]

    means = sums / jnp.maximum(counts_col, 1.0)
    upd = _MOM * avg + (1.0 - _MOM) * means
    updn = upd / jnp.maximum(jnp.sqrt(jnp.sum(upd * upd, axis=1, keepdims=True)), 1e-12)
    new_avg = jnp.where(counts_col > 0.0, updn, avg)

    simil = lax.dot_general(proj_ref[...], new_avg, (((1,), (1,)), ((), ())),
                            preferred_element_type=jnp.float32)
    class_sims = jnp.max(simil, axis=1)
    ids_ref[...] = jnp.argmax(simil, axis=1).astype(jnp.int32).reshape(1, _T)

    pad_ref[0, pl.ds(_PAD, _T)] = class_sims
    for k in range(_PAD):
        pad_ref[0, k:k + 1] = pad_ref[0, 2 * _PAD - k:2 * _PAD - k + 1]
        pad_ref[0, _PAD + _T + k:_PAD + _T + k + 1] = \
            pad_ref[0, _PAD + _T - 2 - k:_PAD + _T - 1 - k]
    acc = pad_ref[0, pl.ds(0, _T)]
    for j in range(1, _KS):
        acc = acc + pad_ref[0, pl.ds(j, _T)]
    sm_ref[...] = (acc * (1.0 / _KS)).reshape(1, _T)

    any_mask = jnp.sum(counts_col) > 0.0
    newproj_ref[...] = jnp.where(any_mask,
                                 _MOM * w_ref[...] + (1.0 - _MOM) * cov_ref[...],
                                 w_ref[...])


@jax.jit
def kernel(features, proj_w, avg_features):
    proj, preds3d = pl.pallas_call(
        _stage1_body,
        grid=(_NBLK,),
        in_specs=[
            pl.BlockSpec((_BLK, _D), lambda i: (i, 0)),
            pl.BlockSpec((_K, _D), lambda i: (0, 0)),
            pl.BlockSpec((_C, _K), lambda i: (0, 0)),
        ],
        out_specs=[
            pl.BlockSpec((_BLK, _K), lambda i: (i, 0)),
            pl.BlockSpec((1, 1, _BLK), lambda i: (i, 0, 0)),
        ],
        out_shape=[
            jax.ShapeDtypeStruct((_T, _K), jnp.float32),
            jax.ShapeDtypeStruct((_NBLK, 1, _BLK), jnp.int32),
        ],
    )(features, proj_w, avg_features)

    preds = preds3d.reshape(_T)
    zsum = jnp.zeros((_CD * _K,), jnp.float32)
    sums = _sc_segsum(proj, preds, zsum).reshape(_NW, _CD, _K)

    cov = pl.pallas_call(
        _cov_body,
        grid=(_NBLK,),
        in_specs=[pl.BlockSpec((_BLK, _D), lambda i: (i, 0))],
        out_specs=pl.BlockSpec((_K, _D), lambda i: (0, 0)),
        out_shape=jax.ShapeDtypeStruct((_K, _D), jnp.float32),
    )(features)

    smoothed, class_ids, new_proj = pl.pallas_call(
        _stage2_body,
        out_shape=[
            jax.ShapeDtypeStruct((1, _T), jnp.float32),
            jax.ShapeDtypeStruct((1, _T), jnp.int32),
            jax.ShapeDtypeStruct((_K, _D), jnp.float32),
        ],
        scratch_shapes=[pltpu.VMEM((1, _T + 2 * _PAD), jnp.float32)],
    )(proj, sums, preds.reshape(1, _T), avg_features, proj_w, cov)

    return smoothed.reshape(_T), class_ids.reshape(_T), new_proj
